# baseline (speedup 1.0000x reference)
import functools

import jax
import jax.numpy as jnp
from jax import lax
from jax.experimental import pallas as pl
from jax.experimental.pallas import tpu as pltpu
from jax.experimental.pallas import tpu_sc as plsc

NUM_BUCKETS = 1000000
DIM = 64
DIM_PAD = 128
B_TOTAL = 4096 * 200

NC = 2
NS = 16
NW = NC * NS

SUB = 4
MACRO = DIM_PAD * SUB
N_MACRO = NUM_BUCKETS // MACRO
TAIL_START = N_MACRO * MACRO
M_ITERS = (N_MACRO + NW - 1) // NW

B_PER_W = B_TOTAL // NW
NBUF = 4
CHUNK = 160
N_CHUNKS = B_PER_W // CHUNK
N_ROUNDS = N_CHUNKS // NBUF


@functools.partial(
    pl.kernel,
    out_type=jax.ShapeDtypeStruct((NUM_BUCKETS, DIM_PAD), jnp.float32),
    mesh=plsc.VectorSubcoreMesh(core_axis_name="c", subcore_axis_name="s"),
    scratch_types=[
        *[pltpu.VMEM((DIM, MACRO), jnp.float32) for _ in range(2)],
        *[pltpu.VMEM((DIM_PAD, DIM_PAD), jnp.float32) for _ in range(2)],
        *[pltpu.SemaphoreType.DMA for _ in range(4)],
    ],
    compiler_params=pltpu.CompilerParams(
        use_tc_tiling_on_sc=True,
        needs_layout_passes=False,
        disable_bounds_checks=True,
    ),
)
def _transpose_kernel(wt_hbm, tail_hbm, out_hbm, in0, in1, t0, t1, *sems):
    ins = (in0, in1)
    outs = (t0, t1)
    isem = sems[:2]
    osem = sems[2:]
    wid = lax.axis_index("s") * NC + lax.axis_index("c")

    def blk(m):
        return wid + NW * m

    def start_load(m, b):
        pltpu.async_copy(
            wt_hbm.at[:, pl.ds(blk(m) * MACRO, MACRO)], ins[b], isem[b]
        )

    def wait_load(b):
        pltpu.make_async_copy(
            wt_hbm.at[:, pl.ds(0, MACRO)], ins[b], isem[b]
        ).wait()

    def start_store(m, s, sb):
        pltpu.async_copy(
            outs[sb],
            out_hbm.at[pl.ds(blk(m) * MACRO + s * DIM_PAD, DIM_PAD)],
            osem[sb],
        )

    def wait_store(sb):
        pltpu.make_async_copy(
            outs[sb], out_hbm.at[pl.ds(0, DIM_PAD)], osem[sb]
        ).wait()

    lane = lax.iota(jnp.int32, 16)
    rows16 = [lane + 16 * k for k in range(DIM // 16)]

    def transpose_block(b, s, sb):
        @plsc.parallel_loop(0, DIM_PAD, unroll=8)
        def _(u):
            col = jnp.full((16,), s * DIM_PAD, jnp.int32) + u
            for k in range(DIM // 16):
                vals = plsc.load_gather(ins[b], [rows16[k], col])
                outs[sb][u, pl.ds(16 * k, 16)] = vals

    for b in range(2):
        start_load(b, b)

    def m_body(m, b, carry):
        @pl.when(blk(m) < N_MACRO)
        def _():
            wait_load(b)
            for s in range(SUB):
                sb = s % 2

                @pl.when((m > 0) | (s >= 2))
                def _():
                    wait_store(sb)

                transpose_block(b, s, sb)
                start_store(m, s, sb)

            @pl.when(blk(m + 2) < N_MACRO)
            def _():
                start_load(m + 2, b)

        return carry

    lax.fori_loop(
        0, M_ITERS // 2,
        lambda r, c: m_body(2 * r + 1, 1, m_body(2 * r, 0, c)), 0,
    )
    if M_ITERS % 2:
        m_body(M_ITERS - 1, (M_ITERS - 1) % 2, 0)
    wait_store(0)
    wait_store(1)

    @pl.when(wid == 0)
    def _():
        pltpu.sync_copy(tail_hbm, outs[0].at[pl.ds(0, DIM)])
        pltpu.sync_copy(outs[0].at[pl.ds(0, DIM)], out_hbm.at[pl.ds(TAIL_START, DIM)])


@functools.partial(
    pl.kernel,
    out_type=jax.ShapeDtypeStruct((B_TOTAL, DIM_PAD), jnp.float32),
    mesh=plsc.VectorSubcoreMesh(core_axis_name="c", subcore_axis_name="s"),
    scratch_types=[
        pltpu.VMEM((B_PER_W,), jnp.int32),
        *[pltpu.VMEM((CHUNK, DIM_PAD), jnp.float32) for _ in range(NBUF)],
        *[pltpu.SemaphoreType.DMA for _ in range(2 * NBUF)],
    ],
    compiler_params=pltpu.CompilerParams(use_tc_tiling_on_sc=True),
)
def _gather_kernel(idx_hbm, table_hbm, out_hbm, idx_v, *bufs_and_sems):
    rows = bufs_and_sems[:NBUF]
    gsem = bufs_and_sems[NBUF:2 * NBUF]
    ssem = bufs_and_sems[2 * NBUF:]
    wid = lax.axis_index("s") * NC + lax.axis_index("c")
    base = wid * B_PER_W
    pltpu.sync_copy(idx_hbm.at[pl.ds(base, B_PER_W)], idx_v)

    def start_gather(g, b):
        pltpu.async_copy(
            table_hbm.at[idx_v.at[pl.ds(g * CHUNK, CHUNK)]], rows[b], gsem[b]
        )

    def wait_gather(b):
        pltpu.make_async_copy(
            table_hbm.at[idx_v.at[pl.ds(0, CHUNK)]], rows[b], gsem[b]
        ).wait()

    def wait_store(b):
        pltpu.make_async_copy(
            rows[b], out_hbm.at[pl.ds(base, CHUNK)], ssem[b]
        ).wait()

    for b in range(NBUF - 1):
        start_gather(b, b)

    def round_body(r, carry):
        for b in range(NBUF):
            g = r * NBUF + b
            wait_gather(b)
            pltpu.async_copy(
                rows[b], out_hbm.at[pl.ds(base + g * CHUNK, CHUNK)], ssem[b]
            )
            pb = (b - 1) % NBUF
            if b > 0:
                wait_store(pb)
            else:
                @pl.when(r > 0)
                def _():
                    wait_store(pb)

            @pl.when(g + NBUF - 1 < N_CHUNKS)
            def _():
                start_gather(g + NBUF - 1, pb)
        return carry

    lax.fori_loop(0, N_ROUNDS, round_body, 0)
    wait_store((N_CHUNKS - 1) % NBUF)


def kernel(token_ids, weight):
    idx = jnp.reshape(token_ids, (B_TOTAL,)).astype(jnp.int32)
    wt = weight.T
    tail = jnp.pad(weight[TAIL_START:], ((0, 0), (0, DIM_PAD - DIM)))
    w128 = _transpose_kernel(wt, tail)
    out = _gather_kernel(idx, w128)
    return jnp.reshape(out[:, :DIM], (*token_ids.shape, DIM))


# speedup vs baseline: 1.2613x; 1.2613x over previous
import functools

import jax
import jax.numpy as jnp
from jax import lax
from jax.experimental import pallas as pl
from jax.experimental.pallas import tpu as pltpu
from jax.experimental.pallas import tpu_sc as plsc

NUM_BUCKETS = 1000000
DIM = 64
DIM_PAD = 128
B_TOTAL = 4096 * 200

NC = 2
NS = 16
NW = NC * NS
B_PER_W = B_TOTAL // NW
NBUF = 2
CHUNK = 320
N_CHUNKS = B_PER_W // CHUNK
N_ROUNDS = N_CHUNKS // NBUF


@functools.partial(
    pl.kernel,
    out_type=jax.ShapeDtypeStruct((B_TOTAL, DIM_PAD), jnp.float32),
    mesh=plsc.VectorSubcoreMesh(core_axis_name="c", subcore_axis_name="s"),
    scratch_types=[
        pltpu.VMEM((B_PER_W,), jnp.int32),
        *[pltpu.VMEM((CHUNK, DIM_PAD), jnp.float32) for _ in range(NBUF)],
        *[pltpu.SemaphoreType.DMA for _ in range(2 * NBUF)],
    ],
    compiler_params=pltpu.CompilerParams(use_tc_tiling_on_sc=True),
)
def _gather_kernel(idx_hbm, table_hbm, out_hbm, idx_v, *bufs_and_sems):
    rows = bufs_and_sems[:NBUF]
    gsem = bufs_and_sems[NBUF:2 * NBUF]
    ssem = bufs_and_sems[2 * NBUF:]
    wid = lax.axis_index("s") * NC + lax.axis_index("c")
    base = wid * B_PER_W
    pltpu.sync_copy(idx_hbm.at[pl.ds(base, B_PER_W)], idx_v)

    def start_gather(g, b):
        pltpu.async_copy(
            table_hbm.at[idx_v.at[pl.ds(g * CHUNK, CHUNK)]], rows[b], gsem[b]
        )

    def wait_gather(b):
        pltpu.make_async_copy(
            table_hbm.at[idx_v.at[pl.ds(0, CHUNK)]], rows[b], gsem[b]
        ).wait()

    def wait_store(b):
        pltpu.make_async_copy(
            rows[b], out_hbm.at[pl.ds(base, CHUNK)], ssem[b]
        ).wait()

    for b in range(NBUF - 1):
        start_gather(b, b)

    def round_body(r, carry):
        for b in range(NBUF):
            g = r * NBUF + b
            wait_gather(b)
            pltpu.async_copy(
                rows[b], out_hbm.at[pl.ds(base + g * CHUNK, CHUNK)], ssem[b]
            )
            pb = (b - 1) % NBUF
            if b > 0:
                wait_store(pb)
            else:
                @pl.when(r > 0)
                def _():
                    wait_store(pb)

            @pl.when(g + NBUF - 1 < N_CHUNKS)
            def _():
                start_gather(g + NBUF - 1, pb)
        return carry

    lax.fori_loop(0, N_ROUNDS, round_body, 0)
    wait_store((N_CHUNKS - 1) % NBUF)


def kernel(token_ids, weight):
    idx = jnp.reshape(token_ids, (B_TOTAL,)).astype(jnp.int32)
    w128 = jnp.pad(weight, ((0, 0), (0, DIM_PAD - DIM)))
    out = _gather_kernel(idx, w128)
    return jnp.reshape(out[:, :DIM], (*token_ids.shape, DIM))


# speedup vs baseline: 1.2644x; 1.0025x over previous
import functools

import jax
import jax.numpy as jnp
from jax import lax
from jax.experimental import pallas as pl
from jax.experimental.pallas import tpu as pltpu
from jax.experimental.pallas import tpu_sc as plsc

NUM_BUCKETS = 1000000
DIM = 64
DIM_PAD = 128
B_TOTAL = 4096 * 200

NC = 2
NS = 16
NW = NC * NS
B_PER_W = B_TOTAL // NW
NBUF = 4
CHUNK = 160
N_CHUNKS = B_PER_W // CHUNK
N_ROUNDS = N_CHUNKS // NBUF


@functools.partial(
    pl.kernel,
    out_type=jax.ShapeDtypeStruct((B_TOTAL, DIM_PAD), jnp.float32),
    mesh=plsc.VectorSubcoreMesh(core_axis_name="c", subcore_axis_name="s"),
    scratch_types=[
        pltpu.VMEM((B_PER_W,), jnp.int32),
        *[pltpu.VMEM((CHUNK, DIM_PAD), jnp.float32) for _ in range(NBUF)],
        *[pltpu.SemaphoreType.DMA for _ in range(2 * NBUF)],
    ],
    compiler_params=pltpu.CompilerParams(use_tc_tiling_on_sc=True),
)
def _gather_kernel(idx_hbm, table_hbm, out_hbm, idx_v, *bufs_and_sems):
    rows = bufs_and_sems[:NBUF]
    gsem = bufs_and_sems[NBUF:2 * NBUF]
    ssem = bufs_and_sems[2 * NBUF:]
    wid = lax.axis_index("s") * NC + lax.axis_index("c")
    base = wid * B_PER_W
    pltpu.sync_copy(idx_hbm.at[pl.ds(base, B_PER_W)], idx_v)

    def start_gather(g, b):
        pltpu.async_copy(
            table_hbm.at[idx_v.at[pl.ds(g * CHUNK, CHUNK)]], rows[b], gsem[b]
        )

    def wait_gather(b):
        pltpu.make_async_copy(
            table_hbm.at[idx_v.at[pl.ds(0, CHUNK)]], rows[b], gsem[b]
        ).wait()

    def wait_store(b):
        pltpu.make_async_copy(
            rows[b], out_hbm.at[pl.ds(base, CHUNK)], ssem[b]
        ).wait()

    for b in range(NBUF - 1):
        start_gather(b, b)

    def round_body(r, carry):
        for b in range(NBUF):
            g = r * NBUF + b
            wait_gather(b)
            pltpu.async_copy(
                rows[b], out_hbm.at[pl.ds(base + g * CHUNK, CHUNK)], ssem[b]
            )
            pb = (b - 1) % NBUF
            if b > 0:
                wait_store(pb)
            else:
                @pl.when(r > 0)
                def _():
                    wait_store(pb)

            @pl.when(g + NBUF - 1 < N_CHUNKS)
            def _():
                start_gather(g + NBUF - 1, pb)
        return carry

    lax.fori_loop(0, N_ROUNDS, round_body, 0)
    wait_store((N_CHUNKS - 1) % NBUF)


def kernel(token_ids, weight):
    idx = jnp.reshape(token_ids, (B_TOTAL,)).astype(jnp.int32)
    w128 = jnp.pad(weight, ((0, 0), (0, DIM_PAD - DIM)))
    out = _gather_kernel(idx, w128)
    return jnp.reshape(out[:, :DIM], (*token_ids.shape, DIM))
